# baseline (speedup 1.0000x reference)
import functools

import jax
import jax.numpy as jnp
from jax import lax
from jax.experimental import pallas as pl
from jax.experimental.pallas import tpu as pltpu
from jax.experimental.pallas import tpu_sc as plsc

NC = 2
NS = 16
NW = NC * NS
L = 16

N = 10000
D = 128
NPAD = 10240
F = D // NW
TRASH = F * NPAD

C = 6400

NEG = float("-inf")


def _sc_body(xt_hbm, src_hbm, dst_hbm, mt_hbm, acc, xs, src_c0, src_c1,
             dst_c0, dst_c1, sem_x, sem_s, sem_d):
    src_c = (src_c0, src_c1)
    dst_c = (dst_c0, dst_c1)
    cid = lax.axis_index("c")
    sid = lax.axis_index("s")
    wid = sid * NC + cid

    E = src_hbm.shape[0]
    n_chunks = E // C

    xcp = pltpu.async_copy(
        xt_hbm.at[pl.ds(wid * F * NPAD, F * NPAD)], xs, sem_x)

    def chunk_off(ci):
        f = ci + wid * (n_chunks // NW)
        return jnp.where(f >= n_chunks, f - n_chunks, f) * C

    b0 = chunk_off(0)
    pltpu.async_copy(src_hbm.at[pl.ds(b0, C)], src_c[0], sem_s)
    pltpu.async_copy(dst_hbm.at[pl.ds(b0, C)], dst_c[0], sem_d)

    neg_vec = jnp.full((L,), NEG, jnp.float32)

    def init_body(i, _):
        acc[pl.ds(i * L, L)] = neg_vec
        return 0

    lax.fori_loop(0, (F * NPAD + L) // L, init_body, 0)
    xcp.wait()

    lane = lax.iota(jnp.int32, L)
    seg_idx = [jnp.maximum(lane - k, 0) for k in (1, 2, 4, 8)]
    idx_dn = jnp.minimum(lane + 1, L - 1)
    lt_last = lane < (L - 1)
    trash_vec = jnp.full((L,), N, jnp.int32)

    def sort_group(g, slot):
        s = src_c[slot][pl.ds(g * L, L)]
        d = dst_c[slot][pl.ds(g * L, L)]
        ksu, ss = plsc.sort_key_val(plsc.bitcast(d, jnp.uint32), s)
        return plsc.bitcast(ksu, jnp.int32), ss

    def process_group(ks, ss):
        segi = [jnp.where(ks.at[iu].get(mode="promise_in_bounds") == ks,
                          iu, lane)
                for iu in seg_idx]
        nxt = ks.at[idx_dn].get(mode="promise_in_bounds")
        not_last = (nxt == ks) & lt_last
        pos0 = jnp.where(not_last, trash_vec, ks)
        avs = [plsc.load_gather(acc.at[pl.ds(f * NPAD, NPAD)], [ks])
               for f in range(F)]
        mvs = []
        for f in range(F):
            xv = plsc.load_gather(xs.at[pl.ds(f * NPAD, NPAD)], [ss])
            for si in segi:
                xv = jnp.maximum(xv, xv.at[si].get(mode="promise_in_bounds"))
            mvs.append(jnp.maximum(xv, avs[f]))
        for f in range(F):
            plsc.store_scatter(acc.at[pl.ds(f * NPAD, NPAD)], [pos0], mvs[f])

    def process_chunk(ci, slot):
        @pl.when(ci + 1 < n_chunks)
        def _():
            nb = chunk_off(ci + 1)
            pltpu.async_copy(src_hbm.at[pl.ds(nb, C)], src_c[1 - slot],
                             sem_s)
            pltpu.async_copy(dst_hbm.at[pl.ds(nb, C)], dst_c[1 - slot],
                             sem_d)

        cb = chunk_off(ci)
        pltpu.make_async_copy(src_hbm.at[pl.ds(cb, C)], src_c[slot],
                              sem_s).wait()
        pltpu.make_async_copy(dst_hbm.at[pl.ds(cb, C)], dst_c[slot],
                              sem_d).wait()

        def scan_body(i, _):
            g0 = i * 8
            kv = sort_group(g0, slot)
            nxt = sort_group(g0 + 1, slot)
            for u in range(2, 8):
                process_group(*kv)
                kv = nxt
                nxt = sort_group(g0 + u, slot)
            process_group(*kv)
            process_group(*nxt)
            return 0

        lax.fori_loop(0, C // L // 8, scan_body, 0)

    def chunk_pair(o, _):
        process_chunk(o * 2, 0)
        process_chunk(o * 2 + 1, 1)
        return 0

    lax.fori_loop(0, n_chunks // 2, chunk_pair, 0)

    pltpu.sync_copy(acc.at[pl.ds(0, F * NPAD)],
                    mt_hbm.at[pl.ds(wid * F * NPAD, F * NPAD)])


def _sc_segmax(xt_flat, src, dst):
    mesh = plsc.VectorSubcoreMesh(core_axis_name="c", subcore_axis_name="s")
    f = pl.kernel(
        _sc_body,
        out_type=jax.ShapeDtypeStruct((D * NPAD,), jnp.float32),
        mesh=mesh,
        scratch_types=[
            pltpu.VMEM((F * NPAD + L,), jnp.float32),
            pltpu.VMEM((F * NPAD,), jnp.float32),
            pltpu.VMEM((C,), jnp.int32),
            pltpu.VMEM((C,), jnp.int32),
            pltpu.VMEM((C,), jnp.int32),
            pltpu.VMEM((C,), jnp.int32),
            pltpu.SemaphoreType.DMA,
            pltpu.SemaphoreType.DMA,
            pltpu.SemaphoreType.DMA,
        ],
        compiler_params=pltpu.CompilerParams(needs_layout_passes=False),
    )
    return f(xt_flat, src, dst)


BLK = 1024


def _tc_mm1_body(x_ref, w1_ref, b_ref, o_ref):
    o_ref[...] = lax.dot_general(
        x_ref[...], w1_ref[...], (((1,), (1,)), ((), ())),
        preferred_element_type=jnp.float32) + b_ref[...]


def _tc_mm1(xp, W1, b):
    return pl.pallas_call(
        _tc_mm1_body,
        grid=(NPAD // BLK,),
        in_specs=[
            pl.BlockSpec((BLK, D), lambda i: (i, 0)),
            pl.BlockSpec((D, D), lambda i: (0, 0)),
            pl.BlockSpec((1, D), lambda i: (0, 0)),
        ],
        out_specs=pl.BlockSpec((BLK, D), lambda i: (i, 0)),
        out_shape=jax.ShapeDtypeStruct((NPAD, D), jnp.float32),
    )(xp, W1, b)


def _tc_mm2_body(mm1_ref, xt_ref, mt_ref, w2_ref, o_ref):
    mdt = jnp.where(mt_ref[...] > NEG, mt_ref[...] - xt_ref[...],
                    jnp.float32(0.0))
    o_ref[...] = mm1_ref[...] + lax.dot_general(
        mdt, w2_ref[...], (((0,), (1,)), ((), ())),
        preferred_element_type=jnp.float32)


def _tc_mm2(mm1, xtp, mt, W2):
    return pl.pallas_call(
        _tc_mm2_body,
        grid=(NPAD // BLK,),
        in_specs=[
            pl.BlockSpec((BLK, D), lambda i: (i, 0)),
            pl.BlockSpec((D, BLK), lambda i: (0, i)),
            pl.BlockSpec((D, BLK), lambda i: (0, i)),
            pl.BlockSpec((D, D), lambda i: (0, 0)),
        ],
        out_specs=pl.BlockSpec((BLK, D), lambda i: (i, 0)),
        out_shape=jax.ShapeDtypeStruct((NPAD, D), jnp.float32),
    )(mm1, xtp, mt, W2)


def kernel(x, edge_index, W, b):
    src = edge_index[0]
    dst = edge_index[1]
    xtp = jnp.pad(x.T, ((0, 0), (0, NPAD - N)))
    mt_flat = _sc_segmax(xtp.reshape(-1), src, dst)
    mt = mt_flat.reshape(D, NPAD)
    xp = jnp.pad(x, ((0, NPAD - N), (0, 0)))
    mm1 = _tc_mm1(xp, W[:, :D], b.reshape(1, D))
    out = _tc_mm2(mm1, xtp, mt, W[:, D:])
    return out[:N]
